# initial kernel scaffold (unmeasured)
import jax
import jax.numpy as jnp
from jax import lax
from jax.experimental import pallas as pl
from jax.experimental.pallas import tpu as pltpu

P = 16


def kernel(x, w_mat):
    m_per, k = x.shape
    _, n = w_mat.shape
    n_per = n // P
    m = m_per * P

    def body(pos_ref, x_ref, w_ref, out_ref, xb_ref, comm_ref, recv_ref,
             send_sems, recv_sems):
        s = pl.program_id(0)
        my_pos = pos_ref[0]
        j = lax.rem(my_pos + s, P)

        @pl.when(s == 0)
        def _():
            xb_ref[...] = x_ref[...].astype(jnp.bfloat16)

        y = jnp.dot(
            xb_ref[...],
            w_ref[...].astype(jnp.bfloat16),
            preferred_element_type=jnp.float32,
        )
        yb = y.astype(jnp.bfloat16)

        @pl.when(s == 0)
        def _():
            recv_ref[pl.ds(my_pos * m_per, m_per), :] = yb

        @pl.when(s > 0)
        def _():
            comm_ref[pl.ds(j * m_per, m_per), :] = yb
            rdma = pltpu.make_async_remote_copy(
                src_ref=comm_ref.at[pl.ds(j * m_per, m_per), :],
                dst_ref=recv_ref.at[pl.ds(my_pos * m_per, m_per), :],
                send_sem=send_sems.at[j],
                recv_sem=recv_sems.at[my_pos],
                device_id=(j,),
                device_id_type=pl.DeviceIdType.MESH,
            )
            rdma.start()

        @pl.when(s == P - 1)
        def _():
            for i in range(P):
                @pl.when(my_pos != i)
                def _(i=i):
                    d = pltpu.make_async_remote_copy(
                        src_ref=comm_ref.at[pl.ds(i * m_per, m_per), :],
                        dst_ref=recv_ref.at[pl.ds(i * m_per, m_per), :],
                        send_sem=send_sems.at[i],
                        recv_sem=recv_sems.at[i],
                        device_id=(i,),
                        device_id_type=pl.DeviceIdType.MESH,
                    )
                    d.wait()
            for i in range(P):
                yi = recv_ref[i * m_per:(i + 1) * m_per, :].astype(jnp.float32)
                out_ref[pl.ds(i * m_per, m_per), :] = yi * (
                    1.0 / (1.0 + jnp.exp(-yi)))

    grid_spec = pltpu.PrefetchScalarGridSpec(
        num_scalar_prefetch=1,
        grid=(P,),
        in_specs=[
            pl.BlockSpec((m_per, k), lambda s, pos: (0, 0)),
            pl.BlockSpec((k, n_per), lambda s, pos: (0, lax.rem(pos[0] + s, P))),
        ],
        out_specs=pl.BlockSpec((m, n_per), lambda s, pos: (0, 0)),
        scratch_shapes=[
            pltpu.VMEM((m_per, k), jnp.bfloat16),
            pltpu.VMEM((m, n_per), jnp.bfloat16),
            pltpu.VMEM((m, n_per), jnp.bfloat16),
            pltpu.SemaphoreType.DMA((P,)),
            pltpu.SemaphoreType.DMA((P,)),
        ],
    )

    my_pos = jnp.full((1,), lax.axis_index("i"), jnp.int32)
    return pl.pallas_call(
        body,
        grid_spec=grid_spec,
        out_shape=jax.ShapeDtypeStruct((m, n_per), jnp.float32),
        compiler_params=pltpu.CompilerParams(
            dimension_semantics=("arbitrary",),
        ),
    )(my_pos, x, w_mat)


# baseline (device time: 76701 ns/iter reference)
import jax
import jax.numpy as jnp
from jax import lax
from jax.experimental import pallas as pl
from jax.experimental.pallas import tpu as pltpu

P = 16


def kernel(x, w_mat):
    m_per, k = x.shape
    _, n = w_mat.shape
    n_per = n // P
    m = m_per * P

    def body(pos_ref, x_ref, w_ref, out_ref, xb_ref, comm_ref, recv_ref,
             send_sems, recv_sems):
        s = pl.program_id(0)
        my_pos = pos_ref[0]
        j = lax.rem(my_pos + s, P)

        @pl.when(s == 0)
        def _():
            xb_ref[...] = x_ref[...].astype(jnp.bfloat16)

        y = jnp.dot(
            xb_ref[...],
            w_ref[...].astype(jnp.bfloat16),
            preferred_element_type=jnp.float32,
        )
        yb = y.astype(jnp.bfloat16)

        @pl.when(s == 0)
        def _():
            recv_ref[pl.ds(my_pos * m_per, m_per), :] = yb

        @pl.when(s > 0)
        def _():
            comm_ref[pl.ds(j * m_per, m_per), :] = yb
            rdma = pltpu.make_async_remote_copy(
                src_ref=comm_ref.at[pl.ds(j * m_per, m_per), :],
                dst_ref=recv_ref.at[pl.ds(my_pos * m_per, m_per), :],
                send_sem=send_sems.at[j],
                recv_sem=recv_sems.at[my_pos],
                device_id=j,
                device_id_type=pl.DeviceIdType.LOGICAL,
            )
            rdma.start()

        @pl.when(s == P - 1)
        def _():
            for i in range(P):
                @pl.when(my_pos != i)
                def _(i=i):
                    d = pltpu.make_async_remote_copy(
                        src_ref=comm_ref.at[pl.ds(i * m_per, m_per), :],
                        dst_ref=recv_ref.at[pl.ds(i * m_per, m_per), :],
                        send_sem=send_sems.at[i],
                        recv_sem=recv_sems.at[i],
                        device_id=i,
                        device_id_type=pl.DeviceIdType.LOGICAL,
                    )
                    d.wait()
            for i in range(P):
                yi = recv_ref[i * m_per:(i + 1) * m_per, :].astype(jnp.float32)
                out_ref[pl.ds(i * m_per, m_per), :] = yi * (
                    1.0 / (1.0 + jnp.exp(-yi)))

    grid_spec = pltpu.PrefetchScalarGridSpec(
        num_scalar_prefetch=1,
        grid=(P,),
        in_specs=[
            pl.BlockSpec((m_per, k), lambda s, pos: (0, 0)),
            pl.BlockSpec((k, n_per), lambda s, pos: (0, lax.rem(pos[0] + s, P))),
        ],
        out_specs=pl.BlockSpec((m, n_per), lambda s, pos: (0, 0)),
        scratch_shapes=[
            pltpu.VMEM((m_per, k), jnp.bfloat16),
            pltpu.VMEM((m, n_per), jnp.bfloat16),
            pltpu.VMEM((m, n_per), jnp.bfloat16),
            pltpu.SemaphoreType.DMA((P,)),
            pltpu.SemaphoreType.DMA((P,)),
        ],
    )

    my_pos = jnp.full((1,), lax.axis_index("i"), jnp.int32)
    return pl.pallas_call(
        body,
        grid_spec=grid_spec,
        out_shape=jax.ShapeDtypeStruct((m, n_per), jnp.float32),
        compiler_params=pltpu.CompilerParams(
            dimension_semantics=("arbitrary",),
        ),
    )(my_pos, x, w_mat)
